# baseline (speedup 1.0000x reference)
import functools

import jax
import jax.numpy as jnp
from jax import lax
from jax.experimental import pallas as pl
from jax.experimental.pallas import tpu as pltpu
from jax.experimental.pallas import tpu_sc as plsc

NC = 2
NS = 16
BATCH = 128


G = 2
NBI = 4


def _make_spmm(n, d, e_pad):
  ept = e_pad // NS
  nch = ept // BATCH
  ngrp = nch // G
  cr = 80
  ncopy = n // cr
  maxi = (ncopy + NS - 1) // NS

  mesh = plsc.VectorSubcoreMesh(core_axis_name="c", subcore_axis_name="s")

  @functools.partial(
      pl.kernel,
      out_type=jax.ShapeDtypeStruct((NC, n, d), jnp.float32),
      mesh=mesh,
      compiler_params=pltpu.CompilerParams(use_tc_tiling_on_sc=False),
      scratch_types=[
          pltpu.VMEM((NBI, G, BATCH), jnp.int32),
          pltpu.VMEM((NBI, G, BATCH), jnp.int32),
          pltpu.VMEM((NBI, G, BATCH), jnp.float32),
          [pltpu.VMEM((BATCH, d), jnp.float32) for _ in range(2)],
          pltpu.VMEM_SHARED((n, d), jnp.float32),
          pltpu.VMEM_SHARED((n, d), jnp.float32),
          [pltpu.SemaphoreType.DMA for _ in range(2)],
          [pltpu.SemaphoreType.DMA for _ in range(2)],
          [pltpu.SemaphoreType.DMA for _ in range(NBI)],
      ],
  )
  def spmm(h_hbm, col_hbm, row_hbm, w_hbm, out_hbm,
           col_b, row_b, w_b, rows, h_s, acc, gsem, ssem, isem):
    c = lax.axis_index("c")
    s = lax.axis_index("s")

    def stage(g, bi):
      src = pl.ds(g * G, G)
      pltpu.async_copy(col_hbm.at[s].at[src], col_b.at[bi], isem[bi])
      pltpu.async_copy(row_hbm.at[s].at[src], row_b.at[bi], isem[bi])
      pltpu.async_copy(w_hbm.at[s].at[src], w_b.at[bi], isem[bi])

    def iwait(g, bi):
      src = pl.ds(g * G, G)
      pltpu.make_async_copy(col_hbm.at[s].at[src], col_b.at[bi], isem[bi]).wait()
      pltpu.make_async_copy(row_hbm.at[s].at[src], row_b.at[bi], isem[bi]).wait()
      pltpu.make_async_copy(w_hbm.at[s].at[src], w_b.at[bi], isem[bi]).wait()

    for g in range(min(NBI, ngrp)):
      stage(g, g)

    zvec = jnp.zeros((16,), jnp.float32)

    def zfill(e, carry):
      for f in range(d // 16):
        rows[0][e, pl.ds(f * 16, 16)] = zvec
      return carry

    lax.fori_loop(0, BATCH, zfill, 0)
    for i in range(maxi):
      k = s + NS * i

      @pl.when(k < ncopy)
      def _():
        off = pl.multiple_of(k * cr, 8)
        chunk = pl.ds(off, cr)
        pltpu.sync_copy(h_hbm.at[c].at[chunk], h_s.at[chunk])
        pltpu.sync_copy(rows[0].at[pl.ds(0, cr)], acc.at[chunk])

    plsc.subcore_barrier()

    def gather(b, bi, r):
      pltpu.async_copy(h_s.at[col_b.at[bi, r]], rows[b], gsem[b])

    def gwait(b, bi, r):
      pltpu.make_async_copy(h_s.at[col_b.at[bi, r]], rows[b], gsem[b]).wait()

    def scatter(b, bi, r):
      pltpu.async_copy(rows[b], acc.at[row_b.at[bi, r]], ssem[b], add=True)

    def swait(b, bi, r):
      pltpu.make_async_copy(rows[b], acc.at[row_b.at[bi, r]], ssem[b]).wait()

    def scale(b, bi, r):
      def sbody(g, carry):
        wg = w_b[bi, r, pl.ds(g * 16, 16)]
        for t in range(16):
          wt = wg[t]
          e = g * 16 + t
          for f in range(d // 16):
            sl = pl.ds(f * 16, 16)
            rows[b][e, sl] = rows[b][e, sl] * wt
        return carry

      lax.fori_loop(0, BATCH // 16, sbody, 0)

    iwait(0, 0)
    gather(0, 0, 0)

    def body(sg, carry):
      for p in range(NBI):
        q = sg * NBI + p
        j0 = 2 * q

        @pl.when(j0 >= 1)
        def _():
          swait(1, p - 1 if p > 0 else NBI - 1, 1)

        gather(1, p, 1)
        gwait(0, p, 0)
        scale(0, p, 0)
        scatter(0, p, 0)

        swait(0, p, 0)
        pn = (p + 1) % NBI

        @pl.when(q + 1 < ngrp)
        def _():
          iwait(q + 1, pn)
          gather(0, pn, 0)

        gwait(1, p, 1)
        scale(1, p, 1)
        scatter(1, p, 1)

        @pl.when(q + NBI < ngrp)
        def _():
          stage(q + NBI, p)

      return carry

    lax.fori_loop(0, ngrp // NBI, body, 0)
    swait(1, (ngrp - 1) % NBI, 1)
    plsc.subcore_barrier()

    for i in range(maxi):
      k = s + NS * i

      @pl.when(k < ncopy)
      def _():
        off = pl.multiple_of(k * cr, 8)
        pltpu.sync_copy(acc.at[pl.ds(off, cr)], out_hbm.at[c].at[pl.ds(off, cr)])

  return spmm


def _mm1(x, w0h):
  n, cdim = x.shape
  d = w0h.shape[1]
  bn = 1000

  def body(x_ref, w_ref, o_ref):
    o_ref[0] = lax.dot_general(
        x_ref[...], w_ref[0], (((1,), (1,)), ((), ())),
        preferred_element_type=jnp.float32)

  return pl.pallas_call(
      body,
      grid=(2, n // bn),
      in_specs=[
          pl.BlockSpec((bn, cdim), lambda c, i: (i, 0)),
          pl.BlockSpec((1, d, cdim), lambda c, i: (c, 0, 0)),
      ],
      out_specs=pl.BlockSpec((1, bn, d), lambda c, i: (c, i, 0)),
      out_shape=jax.ShapeDtypeStruct((2, n, d), jnp.float32),
  )(x, w0h)


def _mm2(s1, w1h):
  _, n, hhalf = s1.shape
  fh = w1h.shape[1]
  bn = 1000

  def body(s_ref, w_ref, o_ref):
    a = jnp.maximum(s_ref[0], 0.0)
    b = jnp.maximum(s_ref[1], 0.0)
    w = w_ref[0]
    o_ref[0] = (
        lax.dot_general(a, w[:, :hhalf], (((1,), (1,)), ((), ())),
                        preferred_element_type=jnp.float32)
        + lax.dot_general(b, w[:, hhalf:], (((1,), (1,)), ((), ())),
                          preferred_element_type=jnp.float32))

  return pl.pallas_call(
      body,
      grid=(2, n // bn),
      in_specs=[
          pl.BlockSpec((2, bn, hhalf), lambda c, i: (0, i, 0)),
          pl.BlockSpec((1, fh, 2 * hhalf), lambda c, i: (c, 0, 0)),
      ],
      out_specs=pl.BlockSpec((1, bn, fh), lambda c, i: (c, i, 0)),
      out_shape=jax.ShapeDtypeStruct((2, n, fh), jnp.float32),
  )(s1, w1h)


def _softmax(s2):
  _, n, fh = s2.shape
  bn = 1000

  def body(s_ref, o_ref):
    z = jnp.concatenate([s_ref[0], s_ref[1]], axis=-1)
    m = jnp.max(z, axis=-1, keepdims=True)
    ez = jnp.exp(z - m)
    o_ref[...] = ez / jnp.sum(ez, axis=-1, keepdims=True)

  return pl.pallas_call(
      body,
      grid=(n // bn,),
      in_specs=[pl.BlockSpec((2, bn, fh), lambda i: (0, i, 0))],
      out_specs=pl.BlockSpec((bn, 2 * fh), lambda i: (i, 0)),
      out_shape=jax.ShapeDtypeStruct((n, 2 * fh), jnp.float32),
  )(s2)


def kernel(X, edge_index, edge_weight, W0, W1):
  n, cdim = X.shape
  hdim = W0.shape[0]
  fdim = W1.shape[0]
  e = edge_weight.shape[0]

  gran = G * NBI * NS * BATCH
  e_pad = ((e + gran - 1) // gran) * gran
  pad = e_pad - e
  nch = e_pad // (NS * BATCH)
  row = jnp.pad(edge_index[0], (0, pad)).reshape(NS, nch, BATCH)
  col = jnp.pad(edge_index[1], (0, pad)).reshape(NS, nch, BATCH)
  w = jnp.pad(edge_weight, (0, pad)).reshape(NS, nch, BATCH)

  w0h = W0.reshape(NC, hdim // NC, cdim)
  h1 = _mm1(X, w0h)
  s1 = _make_spmm(n, hdim // NC, e_pad)(h1, col, row, w)
  w1h = W1.reshape(NC, fdim // NC, hdim)
  z = _mm2(s1, w1h)
  s2 = _make_spmm(n, fdim // NC, e_pad)(z, col, row, w)
  return _softmax(s2)


# speedup vs baseline: 1.0992x; 1.0992x over previous
import functools

import jax
import jax.numpy as jnp
from jax import lax
from jax.experimental import pallas as pl
from jax.experimental.pallas import tpu as pltpu
from jax.experimental.pallas import tpu_sc as plsc

NC = 2
NS = 16
BATCH = 128


G = 3
NBI = 3
RING = 3
BLK = 9


def _make_spmm(n, d, e_pad):
  ept = e_pad // NS
  nch = ept // BATCH
  ngrp = nch // G
  nblk = nch // BLK
  cr = 80
  ncopy = n // cr
  maxi = (ncopy + NS - 1) // NS

  mesh = plsc.VectorSubcoreMesh(core_axis_name="c", subcore_axis_name="s")

  @functools.partial(
      pl.kernel,
      out_type=jax.ShapeDtypeStruct((NC, n, d), jnp.float32),
      mesh=mesh,
      compiler_params=pltpu.CompilerParams(use_tc_tiling_on_sc=False),
      scratch_types=[
          pltpu.VMEM((NBI, G, BATCH), jnp.int32),
          pltpu.VMEM((NBI, G, BATCH), jnp.int32),
          pltpu.VMEM((NBI, G, BATCH), jnp.float32),
          [pltpu.VMEM((BATCH, d), jnp.float32) for _ in range(RING)],
          pltpu.VMEM_SHARED((n, d), jnp.float32),
          pltpu.VMEM_SHARED((n, d), jnp.float32),
          [pltpu.SemaphoreType.DMA for _ in range(RING)],
          [pltpu.SemaphoreType.DMA for _ in range(RING)],
          [pltpu.SemaphoreType.DMA for _ in range(NBI)],
      ],
  )
  def spmm(h_hbm, col_hbm, row_hbm, w_hbm, out_hbm,
           col_b, row_b, w_b, rows, h_s, acc, gsem, ssem, isem):
    c = lax.axis_index("c")
    s = lax.axis_index("s")

    def stage(g, bi):
      src = pl.ds(g * G, G)
      pltpu.async_copy(col_hbm.at[s].at[src], col_b.at[bi], isem[bi])
      pltpu.async_copy(row_hbm.at[s].at[src], row_b.at[bi], isem[bi])
      pltpu.async_copy(w_hbm.at[s].at[src], w_b.at[bi], isem[bi])

    def iwait(g, bi):
      src = pl.ds(g * G, G)
      pltpu.make_async_copy(col_hbm.at[s].at[src], col_b.at[bi], isem[bi]).wait()
      pltpu.make_async_copy(row_hbm.at[s].at[src], row_b.at[bi], isem[bi]).wait()
      pltpu.make_async_copy(w_hbm.at[s].at[src], w_b.at[bi], isem[bi]).wait()

    for g in range(min(NBI, ngrp)):
      stage(g, g)

    zvec = jnp.zeros((16,), jnp.float32)

    def zfill(e, carry):
      for f in range(d // 16):
        rows[0][e, pl.ds(f * 16, 16)] = zvec
      return carry

    lax.fori_loop(0, cr, zfill, 0)
    for i in range(maxi):
      k = s + NS * i

      @pl.when(k < ncopy)
      def _():
        off = pl.multiple_of(k * cr, 8)
        chunk = pl.ds(off, cr)
        pltpu.sync_copy(h_hbm.at[c].at[chunk], h_s.at[chunk])
        pltpu.sync_copy(rows[0].at[pl.ds(0, cr)], acc.at[chunk])

    plsc.subcore_barrier()

    def gather(b, bi, r):
      pltpu.async_copy(h_s.at[col_b.at[bi, r]], rows[b], gsem[b])

    def gwait(b, bi, r):
      pltpu.make_async_copy(h_s.at[col_b.at[bi, r]], rows[b], gsem[b]).wait()

    def scatter(b, bi, r):
      pltpu.async_copy(rows[b], acc.at[row_b.at[bi, r]], ssem[b], add=True)

    def swait(b, bi, r):
      pltpu.make_async_copy(rows[b], acc.at[row_b.at[bi, r]], ssem[b]).wait()

    def scale(b, bi, r):
      def sbody(g, carry):
        wg = w_b[bi, r, pl.ds(g * 16, 16)]
        for t in range(16):
          wt = wg[t]
          e = g * 16 + t
          for f in range(d // 16):
            sl = pl.ds(f * 16, 16)
            rows[b][e, sl] = rows[b][e, sl] * wt
        return carry

      lax.fori_loop(0, BATCH // 16, sbody, 0)

    iwait(0, 0)
    gather(0, 0, 0)
    gather(1, 0, 1)

    def run_chunk(base, u):
      b, p, r = u % RING, (u // G) % NBI, u % G
      gwait(b, p, r)
      scale(b, p, r)
      scatter(b, p, r)
      up = (u - 1) % BLK
      if u == 0:
        @pl.when(base >= 1)
        def _():
          swait(up % RING, (up // G) % NBI, up % G)
      else:
        swait(up % RING, (up // G) % NBI, up % G)
      if u % G == 0:
        g_new = base // G + u // G + 2

        @pl.when(jnp.logical_and(g_new < ngrp, base + u >= 1))
        def _():
          stage(g_new, (u // G + 2) % NBI)
      u2 = (u + 2) % BLK

      def prefetch():
        if u % G == G - 2:
          iwait(base // G + (u + 2) // G, (u2 // G) % NBI)
        gather(u2 % RING, (u2 // G) % NBI, u2 % G)

      if u >= BLK - 2:
        @pl.when(base + BLK < nch)
        def _():
          prefetch()
      else:
        prefetch()

    def body(t, carry):
      base = t * BLK
      for u in range(BLK):
        run_chunk(base, u)
      return carry

    lax.fori_loop(0, nblk, body, 0)
    upl = (nch - 1) % BLK
    swait(upl % RING, (upl // G) % NBI, upl % G)
    plsc.subcore_barrier()

    for i in range(maxi):
      k = s + NS * i

      @pl.when(k < ncopy)
      def _():
        off = pl.multiple_of(k * cr, 8)
        pltpu.sync_copy(acc.at[pl.ds(off, cr)], out_hbm.at[c].at[pl.ds(off, cr)])

  return spmm


def _mm1(x, w0h):
  n, cdim = x.shape
  d = w0h.shape[1]
  bn = 1000

  def body(x_ref, w_ref, o_ref):
    o_ref[0] = lax.dot_general(
        x_ref[...], w_ref[0], (((1,), (1,)), ((), ())),
        preferred_element_type=jnp.float32)

  return pl.pallas_call(
      body,
      grid=(2, n // bn),
      in_specs=[
          pl.BlockSpec((bn, cdim), lambda c, i: (i, 0)),
          pl.BlockSpec((1, d, cdim), lambda c, i: (c, 0, 0)),
      ],
      out_specs=pl.BlockSpec((1, bn, d), lambda c, i: (c, i, 0)),
      out_shape=jax.ShapeDtypeStruct((2, n, d), jnp.float32),
  )(x, w0h)


def _mm2(s1, w1h):
  _, n, hhalf = s1.shape
  fh = w1h.shape[1]
  bn = 1000

  def body(s_ref, w_ref, o_ref):
    a = jnp.maximum(s_ref[0], 0.0)
    b = jnp.maximum(s_ref[1], 0.0)
    w = w_ref[0]
    o_ref[0] = (
        lax.dot_general(a, w[:, :hhalf], (((1,), (1,)), ((), ())),
                        preferred_element_type=jnp.float32)
        + lax.dot_general(b, w[:, hhalf:], (((1,), (1,)), ((), ())),
                          preferred_element_type=jnp.float32))

  return pl.pallas_call(
      body,
      grid=(2, n // bn),
      in_specs=[
          pl.BlockSpec((2, bn, hhalf), lambda c, i: (0, i, 0)),
          pl.BlockSpec((1, fh, 2 * hhalf), lambda c, i: (c, 0, 0)),
      ],
      out_specs=pl.BlockSpec((1, bn, fh), lambda c, i: (c, i, 0)),
      out_shape=jax.ShapeDtypeStruct((2, n, fh), jnp.float32),
  )(s1, w1h)


def _softmax(s2):
  _, n, fh = s2.shape
  bn = 1000

  def body(s_ref, o_ref):
    z = jnp.concatenate([s_ref[0], s_ref[1]], axis=-1)
    m = jnp.max(z, axis=-1, keepdims=True)
    ez = jnp.exp(z - m)
    o_ref[...] = ez / jnp.sum(ez, axis=-1, keepdims=True)

  return pl.pallas_call(
      body,
      grid=(n // bn,),
      in_specs=[pl.BlockSpec((2, bn, fh), lambda i: (0, i, 0))],
      out_specs=pl.BlockSpec((bn, 2 * fh), lambda i: (i, 0)),
      out_shape=jax.ShapeDtypeStruct((n, 2 * fh), jnp.float32),
  )(s2)


def kernel(X, edge_index, edge_weight, W0, W1):
  n, cdim = X.shape
  hdim = W0.shape[0]
  fdim = W1.shape[0]
  e = edge_weight.shape[0]

  gran = G * NBI * NS * BATCH
  e_pad = ((e + gran - 1) // gran) * gran
  pad = e_pad - e
  nch = e_pad // (NS * BATCH)
  row = jnp.pad(edge_index[0], (0, pad)).reshape(NS, nch, BATCH)
  col = jnp.pad(edge_index[1], (0, pad)).reshape(NS, nch, BATCH)
  w = jnp.pad(edge_weight, (0, pad)).reshape(NS, nch, BATCH)

  w0h = W0.reshape(NC, hdim // NC, cdim)
  h1 = _mm1(X, w0h)
  s1 = _make_spmm(n, hdim // NC, e_pad)(h1, col, row, w)
  w1h = W1.reshape(NC, fdim // NC, hdim)
  z = _mm2(s1, w1h)
  s2 = _make_spmm(n, fdim // NC, e_pad)(z, col, row, w)
  return _softmax(s2)


# speedup vs baseline: 1.1001x; 1.0009x over previous
import functools

import jax
import jax.numpy as jnp
from jax import lax
from jax.experimental import pallas as pl
from jax.experimental.pallas import tpu as pltpu
from jax.experimental.pallas import tpu_sc as plsc

NC = 2
NS = 16
BATCH = 128


G = 3
NBI = 3
RING = 3
BLK = 9


def _make_spmm(n, d, e_pad):
  ept = e_pad // NS
  nch = ept // BATCH
  ngrp = nch // G
  nblk = nch // BLK
  cr = 80
  ncopy = n // cr
  maxi = (ncopy + NS - 1) // NS

  mesh = plsc.VectorSubcoreMesh(core_axis_name="c", subcore_axis_name="s")

  @functools.partial(
      pl.kernel,
      out_type=jax.ShapeDtypeStruct((NC, n, d), jnp.float32),
      mesh=mesh,
      compiler_params=pltpu.CompilerParams(use_tc_tiling_on_sc=False),
      scratch_types=[
          pltpu.VMEM((NBI, G, BATCH), jnp.int32),
          pltpu.VMEM((NBI, G, BATCH), jnp.int32),
          pltpu.VMEM((NBI, G, BATCH), jnp.float32),
          [pltpu.VMEM((BATCH, d), jnp.float32) for _ in range(RING)],
          pltpu.VMEM_SHARED((n, d), jnp.float32),
          pltpu.VMEM_SHARED((n, d), jnp.float32),
          [pltpu.SemaphoreType.DMA for _ in range(RING)],
          [pltpu.SemaphoreType.DMA for _ in range(RING)],
          [pltpu.SemaphoreType.DMA for _ in range(NBI)],
      ],
  )
  def spmm(h_hbm, col_hbm, row_hbm, w_hbm, out_hbm,
           col_b, row_b, w_b, rows, h_s, acc, gsem, ssem, isem):
    c = lax.axis_index("c")
    s = lax.axis_index("s")

    def stage(g, bi):
      src = pl.ds(g * G, G)
      pltpu.async_copy(col_hbm.at[s].at[src], col_b.at[bi], isem[bi])
      pltpu.async_copy(row_hbm.at[s].at[src], row_b.at[bi], isem[bi])
      pltpu.async_copy(w_hbm.at[s].at[src], w_b.at[bi], isem[bi])

    def iwait(g, bi):
      src = pl.ds(g * G, G)
      pltpu.make_async_copy(col_hbm.at[s].at[src], col_b.at[bi], isem[bi]).wait()
      pltpu.make_async_copy(row_hbm.at[s].at[src], row_b.at[bi], isem[bi]).wait()
      pltpu.make_async_copy(w_hbm.at[s].at[src], w_b.at[bi], isem[bi]).wait()

    for g in range(min(NBI, ngrp)):
      stage(g, g)

    zvec = jnp.zeros((16,), jnp.float32)

    def zfill(e, carry):
      for f in range(d // 16):
        rows[0][e, pl.ds(f * 16, 16)] = zvec
      return carry

    lax.fori_loop(0, cr, zfill, 0)
    for i in range(maxi):
      k = s + NS * i

      @pl.when(k < ncopy)
      def _():
        off = pl.multiple_of(k * cr, 8)
        chunk = pl.ds(off, cr)
        pltpu.sync_copy(h_hbm.at[c].at[chunk], h_s.at[chunk])
        pltpu.sync_copy(rows[0].at[pl.ds(0, cr)], acc.at[chunk])

    plsc.subcore_barrier()

    def gather(b, bi, r):
      pltpu.async_copy(h_s.at[col_b.at[bi, r]], rows[b], gsem[b])

    def gwait(b, bi, r):
      pltpu.make_async_copy(h_s.at[col_b.at[bi, r]], rows[b], gsem[b]).wait()

    def scatter(b, bi, r):
      pltpu.async_copy(rows[b], acc.at[row_b.at[bi, r]], ssem[b], add=True)

    def swait(b, bi, r):
      pltpu.make_async_copy(rows[b], acc.at[row_b.at[bi, r]], ssem[b]).wait()

    bcast_dnums = lax.GatherDimensionNumbers(
        offset_dims=(), collapsed_slice_dims=(0,), start_index_map=(0,))
    bcast_idx = [jnp.full((16, 1), t, jnp.int32) for t in range(16)]

    def scale(b, bi, r):
      def sbody(g, carry):
        wg = w_b[bi, r, pl.ds(g * 16, 16)]
        for t in range(16):
          wt = lax.gather(wg, bcast_idx[t], bcast_dnums, (1,),
                          mode=lax.GatherScatterMode.PROMISE_IN_BOUNDS)
          e = g * 16 + t
          for f in range(d // 16):
            sl = pl.ds(f * 16, 16)
            rows[b][e, sl] = rows[b][e, sl] * wt
        return carry

      lax.fori_loop(0, BATCH // 16, sbody, 0)

    iwait(0, 0)
    gather(0, 0, 0)
    gather(1, 0, 1)

    def run_chunk(base, u):
      b, p, r = u % RING, (u // G) % NBI, u % G
      gwait(b, p, r)
      scale(b, p, r)
      scatter(b, p, r)
      up = (u - 1) % BLK
      if u == 0:
        @pl.when(base >= 1)
        def _():
          swait(up % RING, (up // G) % NBI, up % G)
      else:
        swait(up % RING, (up // G) % NBI, up % G)
      if u % G == 0:
        g_new = base // G + u // G + 2

        @pl.when(jnp.logical_and(g_new < ngrp, base + u >= 1))
        def _():
          stage(g_new, (u // G + 2) % NBI)
      u2 = (u + 2) % BLK

      def prefetch():
        if u % G == G - 2:
          iwait(base // G + (u + 2) // G, (u2 // G) % NBI)
        gather(u2 % RING, (u2 // G) % NBI, u2 % G)

      if u >= BLK - 2:
        @pl.when(base + BLK < nch)
        def _():
          prefetch()
      else:
        prefetch()

    def body(t, carry):
      base = t * BLK
      for u in range(BLK):
        run_chunk(base, u)
      return carry

    lax.fori_loop(0, nblk, body, 0)
    upl = (nch - 1) % BLK
    swait(upl % RING, (upl // G) % NBI, upl % G)
    plsc.subcore_barrier()

    for i in range(maxi):
      k = s + NS * i

      @pl.when(k < ncopy)
      def _():
        off = pl.multiple_of(k * cr, 8)
        pltpu.sync_copy(acc.at[pl.ds(off, cr)], out_hbm.at[c].at[pl.ds(off, cr)])

  return spmm


def _mm1(x, w0h):
  n, cdim = x.shape
  d = w0h.shape[1]
  bn = 1000

  def body(x_ref, w_ref, o_ref):
    o_ref[0] = lax.dot_general(
        x_ref[...], w_ref[0], (((1,), (1,)), ((), ())),
        preferred_element_type=jnp.float32)

  return pl.pallas_call(
      body,
      grid=(2, n // bn),
      in_specs=[
          pl.BlockSpec((bn, cdim), lambda c, i: (i, 0)),
          pl.BlockSpec((1, d, cdim), lambda c, i: (c, 0, 0)),
      ],
      out_specs=pl.BlockSpec((1, bn, d), lambda c, i: (c, i, 0)),
      out_shape=jax.ShapeDtypeStruct((2, n, d), jnp.float32),
  )(x, w0h)


def _mm2(s1, w1h):
  _, n, hhalf = s1.shape
  fh = w1h.shape[1]
  bn = 1000

  def body(s_ref, w_ref, o_ref):
    a = jnp.maximum(s_ref[0], 0.0)
    b = jnp.maximum(s_ref[1], 0.0)
    w = w_ref[0]
    o_ref[0] = (
        lax.dot_general(a, w[:, :hhalf], (((1,), (1,)), ((), ())),
                        preferred_element_type=jnp.float32)
        + lax.dot_general(b, w[:, hhalf:], (((1,), (1,)), ((), ())),
                          preferred_element_type=jnp.float32))

  return pl.pallas_call(
      body,
      grid=(2, n // bn),
      in_specs=[
          pl.BlockSpec((2, bn, hhalf), lambda c, i: (0, i, 0)),
          pl.BlockSpec((1, fh, 2 * hhalf), lambda c, i: (c, 0, 0)),
      ],
      out_specs=pl.BlockSpec((1, bn, fh), lambda c, i: (c, i, 0)),
      out_shape=jax.ShapeDtypeStruct((2, n, fh), jnp.float32),
  )(s1, w1h)


def _softmax(s2):
  _, n, fh = s2.shape
  bn = 1000

  def body(s_ref, o_ref):
    z = jnp.concatenate([s_ref[0], s_ref[1]], axis=-1)
    m = jnp.max(z, axis=-1, keepdims=True)
    ez = jnp.exp(z - m)
    o_ref[...] = ez / jnp.sum(ez, axis=-1, keepdims=True)

  return pl.pallas_call(
      body,
      grid=(n // bn,),
      in_specs=[pl.BlockSpec((2, bn, fh), lambda i: (0, i, 0))],
      out_specs=pl.BlockSpec((bn, 2 * fh), lambda i: (i, 0)),
      out_shape=jax.ShapeDtypeStruct((n, 2 * fh), jnp.float32),
  )(s2)


def kernel(X, edge_index, edge_weight, W0, W1):
  n, cdim = X.shape
  hdim = W0.shape[0]
  fdim = W1.shape[0]
  e = edge_weight.shape[0]

  gran = G * NBI * NS * BATCH
  e_pad = ((e + gran - 1) // gran) * gran
  pad = e_pad - e
  nch = e_pad // (NS * BATCH)
  row = jnp.pad(edge_index[0], (0, pad)).reshape(NS, nch, BATCH)
  col = jnp.pad(edge_index[1], (0, pad)).reshape(NS, nch, BATCH)
  w = jnp.pad(edge_weight, (0, pad)).reshape(NS, nch, BATCH)

  w0h = W0.reshape(NC, hdim // NC, cdim)
  h1 = _mm1(X, w0h)
  s1 = _make_spmm(n, hdim // NC, e_pad)(h1, col, row, w)
  w1h = W1.reshape(NC, fdim // NC, hdim)
  z = _mm2(s1, w1h)
  s2 = _make_spmm(n, fdim // NC, e_pad)(z, col, row, w)
  return _softmax(s2)


# speedup vs baseline: 1.7039x; 1.5489x over previous
import functools

import jax
import jax.numpy as jnp
from jax import lax
from jax.experimental import pallas as pl
from jax.experimental.pallas import tpu as pltpu
from jax.experimental.pallas import tpu_sc as plsc

NC = 2
NS = 16
BATCH = 128


G = 3
NBI = 3
RING = 3
BLK = 9


def _make_spmm(n, d, e_pad):
  ept = e_pad // NS
  nch = ept // BATCH
  ngrp = nch // G
  nblk = nch // BLK
  cr = 80
  ncopy = n // cr
  maxi = (ncopy + NS - 1) // NS

  mesh = plsc.VectorSubcoreMesh(core_axis_name="c", subcore_axis_name="s")

  @functools.partial(
      pl.kernel,
      out_type=jax.ShapeDtypeStruct((NC, n, d), jnp.float32),
      mesh=mesh,
      compiler_params=pltpu.CompilerParams(use_tc_tiling_on_sc=False),
      scratch_types=[
          pltpu.VMEM((NBI, G, BATCH), jnp.int32),
          pltpu.VMEM((NBI, G, BATCH), jnp.int32),
          pltpu.VMEM((NBI, G, BATCH), jnp.float32),
          [pltpu.VMEM((BATCH, d), jnp.float32) for _ in range(RING)],
          pltpu.VMEM_SHARED((n, d), jnp.float32),
          pltpu.VMEM_SHARED((n, d), jnp.float32),
          [pltpu.SemaphoreType.DMA for _ in range(RING)],
          [pltpu.SemaphoreType.DMA for _ in range(RING)],
          [pltpu.SemaphoreType.DMA for _ in range(NBI)],
      ],
  )
  def spmm(h_hbm, col_hbm, row_hbm, w_hbm, out_hbm,
           col_b, row_b, w_b, rows, h_s, acc, gsem, ssem, isem):
    c = lax.axis_index("c")
    s = lax.axis_index("s")

    def stage(g, bi):
      src = pl.ds(g * G, G)
      pltpu.async_copy(col_hbm.at[s].at[src], col_b.at[bi], isem[bi])
      pltpu.async_copy(row_hbm.at[s].at[src], row_b.at[bi], isem[bi])
      pltpu.async_copy(w_hbm.at[s].at[src], w_b.at[bi], isem[bi])

    def iwait(g, bi):
      src = pl.ds(g * G, G)
      pltpu.make_async_copy(col_hbm.at[s].at[src], col_b.at[bi], isem[bi]).wait()
      pltpu.make_async_copy(row_hbm.at[s].at[src], row_b.at[bi], isem[bi]).wait()
      pltpu.make_async_copy(w_hbm.at[s].at[src], w_b.at[bi], isem[bi]).wait()

    for g in range(min(NBI, ngrp)):
      stage(g, g)

    zvec = jnp.zeros((16,), jnp.float32)

    def zfill(e, carry):
      for f in range(d // 16):
        rows[0][e, pl.ds(f * 16, 16)] = zvec
      return carry

    lax.fori_loop(0, cr, zfill, 0)
    for i in range(maxi):
      k = s + NS * i

      @pl.when(k < ncopy)
      def _():
        off = pl.multiple_of(k * cr, 8)
        chunk = pl.ds(off, cr)
        pltpu.sync_copy(h_hbm.at[c].at[chunk], h_s.at[chunk])
        pltpu.sync_copy(rows[0].at[pl.ds(0, cr)], acc.at[chunk])

    plsc.subcore_barrier()

    def gather(b, bi, r):
      pltpu.async_copy(h_s.at[col_b.at[bi, r]], rows[b], gsem[b])

    def gwait(b, bi, r):
      pltpu.make_async_copy(h_s.at[col_b.at[bi, r]], rows[b], gsem[b]).wait()

    def scatter(b, bi, r):
      pltpu.async_copy(rows[b], acc.at[row_b.at[bi, r]], ssem[b], add=True)

    def swait(b, bi, r):
      pltpu.make_async_copy(rows[b], acc.at[row_b.at[bi, r]], ssem[b]).wait()

    bcast_dnums = lax.GatherDimensionNumbers(
        offset_dims=(), collapsed_slice_dims=(0,), start_index_map=(0,))
    bcast_idx = [jnp.full((16, 1), t, jnp.int32) for t in range(16)]

    def scale(b, bi, r):
      @plsc.parallel_loop(0, BATCH // 16, unroll=2)
      def _(g):
        wg = w_b[bi, r, pl.ds(g * 16, 16)]
        for t in range(16):
          wt = lax.gather(wg, bcast_idx[t], bcast_dnums, (1,),
                          mode=lax.GatherScatterMode.PROMISE_IN_BOUNDS)
          e = g * 16 + t
          for f in range(d // 16):
            sl = pl.ds(f * 16, 16)
            rows[b][e, sl] = rows[b][e, sl] * wt

    iwait(0, 0)
    gather(0, 0, 0)
    gather(1, 0, 1)

    def run_chunk(base, u):
      b, p, r = u % RING, (u // G) % NBI, u % G
      gwait(b, p, r)
      scale(b, p, r)
      scatter(b, p, r)
      up = (u - 1) % BLK
      if u == 0:
        @pl.when(base >= 1)
        def _():
          swait(up % RING, (up // G) % NBI, up % G)
      else:
        swait(up % RING, (up // G) % NBI, up % G)
      if u % G == 0:
        g_new = base // G + u // G + 2

        @pl.when(jnp.logical_and(g_new < ngrp, base + u >= 1))
        def _():
          stage(g_new, (u // G + 2) % NBI)
      u2 = (u + 2) % BLK

      def prefetch():
        if u % G == G - 2:
          iwait(base // G + (u + 2) // G, (u2 // G) % NBI)
        gather(u2 % RING, (u2 // G) % NBI, u2 % G)

      if u >= BLK - 2:
        @pl.when(base + BLK < nch)
        def _():
          prefetch()
      else:
        prefetch()

    def body(t, carry):
      base = t * BLK
      for u in range(BLK):
        run_chunk(base, u)
      return carry

    lax.fori_loop(0, nblk, body, 0)
    upl = (nch - 1) % BLK
    swait(upl % RING, (upl // G) % NBI, upl % G)
    plsc.subcore_barrier()

    for i in range(maxi):
      k = s + NS * i

      @pl.when(k < ncopy)
      def _():
        off = pl.multiple_of(k * cr, 8)
        pltpu.sync_copy(acc.at[pl.ds(off, cr)], out_hbm.at[c].at[pl.ds(off, cr)])

  return spmm


def _mm1(x, w0h):
  n, cdim = x.shape
  d = w0h.shape[1]
  bn = 1000

  def body(x_ref, w_ref, o_ref):
    o_ref[0] = lax.dot_general(
        x_ref[...], w_ref[0], (((1,), (1,)), ((), ())),
        preferred_element_type=jnp.float32)

  return pl.pallas_call(
      body,
      grid=(2, n // bn),
      in_specs=[
          pl.BlockSpec((bn, cdim), lambda c, i: (i, 0)),
          pl.BlockSpec((1, d, cdim), lambda c, i: (c, 0, 0)),
      ],
      out_specs=pl.BlockSpec((1, bn, d), lambda c, i: (c, i, 0)),
      out_shape=jax.ShapeDtypeStruct((2, n, d), jnp.float32),
  )(x, w0h)


def _mm2(s1, w1h):
  _, n, hhalf = s1.shape
  fh = w1h.shape[1]
  bn = 1000

  def body(s_ref, w_ref, o_ref):
    a = jnp.maximum(s_ref[0], 0.0)
    b = jnp.maximum(s_ref[1], 0.0)
    w = w_ref[0]
    o_ref[0] = (
        lax.dot_general(a, w[:, :hhalf], (((1,), (1,)), ((), ())),
                        preferred_element_type=jnp.float32)
        + lax.dot_general(b, w[:, hhalf:], (((1,), (1,)), ((), ())),
                          preferred_element_type=jnp.float32))

  return pl.pallas_call(
      body,
      grid=(2, n // bn),
      in_specs=[
          pl.BlockSpec((2, bn, hhalf), lambda c, i: (0, i, 0)),
          pl.BlockSpec((1, fh, 2 * hhalf), lambda c, i: (c, 0, 0)),
      ],
      out_specs=pl.BlockSpec((1, bn, fh), lambda c, i: (c, i, 0)),
      out_shape=jax.ShapeDtypeStruct((2, n, fh), jnp.float32),
  )(s1, w1h)


def _softmax(s2):
  _, n, fh = s2.shape
  bn = 1000

  def body(s_ref, o_ref):
    z = jnp.concatenate([s_ref[0], s_ref[1]], axis=-1)
    m = jnp.max(z, axis=-1, keepdims=True)
    ez = jnp.exp(z - m)
    o_ref[...] = ez / jnp.sum(ez, axis=-1, keepdims=True)

  return pl.pallas_call(
      body,
      grid=(n // bn,),
      in_specs=[pl.BlockSpec((2, bn, fh), lambda i: (0, i, 0))],
      out_specs=pl.BlockSpec((bn, 2 * fh), lambda i: (i, 0)),
      out_shape=jax.ShapeDtypeStruct((n, 2 * fh), jnp.float32),
  )(s2)


def kernel(X, edge_index, edge_weight, W0, W1):
  n, cdim = X.shape
  hdim = W0.shape[0]
  fdim = W1.shape[0]
  e = edge_weight.shape[0]

  gran = G * NBI * NS * BATCH
  e_pad = ((e + gran - 1) // gran) * gran
  pad = e_pad - e
  nch = e_pad // (NS * BATCH)
  row = jnp.pad(edge_index[0], (0, pad)).reshape(NS, nch, BATCH)
  col = jnp.pad(edge_index[1], (0, pad)).reshape(NS, nch, BATCH)
  w = jnp.pad(edge_weight, (0, pad)).reshape(NS, nch, BATCH)

  w0h = W0.reshape(NC, hdim // NC, cdim)
  h1 = _mm1(X, w0h)
  s1 = _make_spmm(n, hdim // NC, e_pad)(h1, col, row, w)
  w1h = W1.reshape(NC, fdim // NC, hdim)
  z = _mm2(s1, w1h)
  s2 = _make_spmm(n, fdim // NC, e_pad)(z, col, row, w)
  return _softmax(s2)
